# baseline (speedup 1.0000x reference)
import functools

import jax
import jax.numpy as jnp
from jax import lax
from jax.experimental import pallas as pl
from jax.experimental.pallas import tpu as pltpu
from jax.experimental.pallas import tpu_sc as plsc

N = 10000
E = 320000
D_IN = 128
D_HID = 256
D_OUT = 128

NC = 2
NS = 16
L = 16
HW = D_IN // NC

DEPTH = 5
BB = 128
NBC = 160
E_PAD = NS * NBC * BB
N_PAD = 10240
RPT = N_PAD // NS


def _make_seg_sum(with_deg):
    mesh = plsc.VectorSubcoreMesh(core_axis_name="c", subcore_axis_name="s")
    out_type = [jax.ShapeDtypeStruct((NC, N_PAD, HW), jnp.float32)]
    if with_deg:
        out_type.append(jax.ShapeDtypeStruct((N_PAD,), jnp.float32))
    scratch = (
        [
            pltpu.VMEM((NBC, BB), jnp.int32),
            pltpu.VMEM((NBC, BB), jnp.int32),
        ]
        + [pltpu.VMEM((BB, HW), jnp.float32) for _ in range(DEPTH)]
        + [
            pltpu.VMEM((BB,), jnp.float32),
            pltpu.VMEM_SHARED((N_PAD, HW), jnp.float32),
            pltpu.VMEM_SHARED((N_PAD,), jnp.float32),
        ]
        + [pltpu.SemaphoreType.DMA for _ in range(2 * DEPTH + 1)]
    )

    def body(tables, srcs, dsts, *rest):
        if with_deg:
            acc_out, deg_out = rest[0], rest[1]
            rest = rest[2:]
        else:
            acc_out = rest[0]
            rest = rest[1:]
        src_v, dst_v = rest[0], rest[1]
        bufs = list(rest[2:2 + DEPTH])
        ones_v, acc_sh, deg_sh = rest[2 + DEPTH:5 + DEPTH]
        gsems = list(rest[5 + DEPTH:5 + 2 * DEPTH])
        ssems = list(rest[5 + 2 * DEPTH:5 + 3 * DEPTH])
        dsem = rest[5 + 3 * DEPTH]
        rows0 = bufs[0]
        cid = lax.axis_index("c")
        sid = lax.axis_index("s")
        base = sid * RPT
        on_core0 = cid == 0

        def zrow(r, carry):
            for c in range(HW // L):
                rows0[r, pl.ds(c * L, L)] = jnp.zeros((L,), jnp.float32)
            return carry

        lax.fori_loop(0, BB, zrow, 0)
        for k in range(RPT // BB):
            pltpu.sync_copy(rows0, acc_sh.at[pl.ds(base + k * BB, BB)])
        if with_deg:
            @pl.when(on_core0)
            def _():
                for k in range(RPT // HW):
                    pltpu.sync_copy(rows0.at[0],
                                    deg_sh.at[pl.ds(base + k * HW, HW)])
                for i in range(BB // L):
                    ones_v[pl.ds(i * L, L)] = jnp.ones((L,), jnp.float32)
        pltpu.sync_copy(srcs.at[sid], src_v)
        pltpu.sync_copy(dsts.at[sid], dst_v)
        plsc.subcore_barrier()

        for b in range(DEPTH):
            pltpu.async_copy(tables.at[cid].at[src_v.at[b]], bufs[b], gsems[b])

        def group(g, carry):
            for b in range(DEPTH):
                j = DEPTH * g + b
                pltpu.make_async_copy(tables.at[cid].at[src_v.at[j]],
                                      bufs[b], gsems[b]).wait()
                pltpu.async_copy(bufs[b], acc_sh.at[dst_v.at[j]], ssems[b],
                                 add=True)
                if with_deg:
                    @pl.when(on_core0)
                    def _():
                        pltpu.async_copy(ones_v, deg_sh.at[dst_v.at[j]], dsem,
                                         add=True)
            for b in range(DEPTH):
                jn = DEPTH * (g + 1) + b

                @pl.when(jn < NBC)
                def _():
                    pltpu.make_async_copy(bufs[b], acc_sh.at[dst_v.at[0]],
                                          ssems[b]).wait()
                    pltpu.async_copy(tables.at[cid].at[src_v.at[jn]],
                                     bufs[b], gsems[b])
            return carry

        lax.fori_loop(0, NBC // DEPTH, group, 0)
        for b in range(DEPTH):
            pltpu.make_async_copy(bufs[b], acc_sh.at[dst_v.at[0]],
                                  ssems[b]).wait()
        if with_deg:
            @pl.when(on_core0)
            def _():
                def drain(i, carry):
                    pltpu.make_async_copy(ones_v, deg_sh.at[dst_v.at[0]],
                                          dsem).wait()
                    return carry

                lax.fori_loop(0, NBC, drain, 0)
        plsc.subcore_barrier()

        pltpu.sync_copy(acc_sh.at[pl.ds(base, RPT)],
                        acc_out.at[cid, pl.ds(base, RPT)])
        if with_deg:
            @pl.when(on_core0)
            def _():
                pltpu.sync_copy(deg_sh.at[pl.ds(base, RPT)],
                                deg_out.at[pl.ds(base, RPT)])

    return functools.partial(
        pl.kernel, mesh=mesh, out_type=out_type, scratch_types=scratch,
        compiler_params=pltpu.CompilerParams(use_tc_tiling_on_sc=False),
    )(body)


_seg_sum_deg = _make_seg_sum(True)
_seg_sum = _make_seg_sum(False)

BM = 2000


def _layer1_body(x_ref, a_ref, d_ref, ws_ref, wn_ref, b_ref, wn2_ref,
                 h_ref, g_ref):
    deg = jnp.maximum(d_ref[...], 1.0)
    hn = a_ref[...] / deg
    h = jnp.dot(x_ref[...], ws_ref[...], preferred_element_type=jnp.float32)
    h = h + jnp.dot(hn, wn_ref[...], preferred_element_type=jnp.float32)
    h = jnp.maximum(h + b_ref[...], 0.0)
    h_ref[...] = h
    g_ref[...] = jnp.dot(h, wn2_ref[...], preferred_element_type=jnp.float32)


def _layer2_body(h_ref, a_ref, d_ref, ws_ref, b_ref, out_ref):
    deg = jnp.maximum(d_ref[...], 1.0)
    hn = a_ref[...] / deg
    out = jnp.dot(h_ref[...], ws_ref[...], preferred_element_type=jnp.float32)
    out_ref[...] = out + hn + b_ref[...]


def _row_spec(w):
    return pl.BlockSpec((BM, w), lambda i: (i, 0))


def _full_spec(r, c):
    return pl.BlockSpec((r, c), lambda i: (0, 0))


def kernel(features, edge_index, W_self1, W_neigh1, b1, W_self2, W_neigh2, b2):
    src = edge_index[0].astype(jnp.int32)
    dst = edge_index[1].astype(jnp.int32)
    pad = E_PAD - E
    src_p = jnp.concatenate([src, jnp.zeros((pad,), jnp.int32)]).reshape(NS, NBC, BB)
    dst_p = jnp.concatenate([dst, jnp.full((pad,), N, jnp.int32)]).reshape(NS, NBC, BB)

    tab1 = features.reshape(N, NC, HW).transpose(1, 0, 2)
    acc1, degp = _seg_sum_deg(tab1, src_p, dst_p)
    a1 = acc1[:, :N].transpose(1, 0, 2).reshape(N, D_IN)
    d = degp[:N, None]

    h, g = pl.pallas_call(
        _layer1_body,
        grid=(N // BM,),
        in_specs=[_row_spec(D_IN), _row_spec(D_IN), _row_spec(1),
                  _full_spec(D_IN, D_HID), _full_spec(D_IN, D_HID),
                  _full_spec(1, D_HID), _full_spec(D_HID, D_OUT)],
        out_specs=[_row_spec(D_HID), _row_spec(D_OUT)],
        out_shape=[jax.ShapeDtypeStruct((N, D_HID), jnp.float32),
                   jax.ShapeDtypeStruct((N, D_OUT), jnp.float32)],
    )(features, a1, d, W_self1, W_neigh1, b1.reshape(1, -1), W_neigh2)

    tab2 = g.reshape(N, NC, HW).transpose(1, 0, 2)
    (acc2,) = _seg_sum(tab2, src_p, dst_p)
    a2 = acc2[:, :N].transpose(1, 0, 2).reshape(N, D_OUT)

    out = pl.pallas_call(
        _layer2_body,
        grid=(N // BM,),
        in_specs=[_row_spec(D_HID), _row_spec(D_OUT), _row_spec(1),
                  _full_spec(D_HID, D_OUT), _full_spec(1, D_OUT)],
        out_specs=_row_spec(D_OUT),
        out_shape=jax.ShapeDtypeStruct((N, D_OUT), jnp.float32),
    )(h, a2, d, W_self2, b2.reshape(1, -1))

    return out


# speedup vs baseline: 1.0159x; 1.0159x over previous
import functools

import jax
import jax.numpy as jnp
from jax import lax
from jax.experimental import pallas as pl
from jax.experimental.pallas import tpu as pltpu
from jax.experimental.pallas import tpu_sc as plsc

N = 10000
E = 320000
D_IN = 128
D_HID = 256
D_OUT = 128

NC = 2
NS = 16
L = 16
HW = D_IN // NC

DEPTH = 4
BB = 128
NBC = 160
E_PAD = NS * NBC * BB
N_PAD = 10240
RPT = N_PAD // NS


def _make_seg_sum(with_deg):
    mesh = plsc.VectorSubcoreMesh(core_axis_name="c", subcore_axis_name="s")
    out_type = [jax.ShapeDtypeStruct((N_PAD, NC, HW), jnp.float32)]
    if with_deg:
        out_type.append(jax.ShapeDtypeStruct((N_PAD,), jnp.float32))
    scratch = (
        [
            pltpu.VMEM((NBC, BB), jnp.int32),
            pltpu.VMEM((NBC, BB), jnp.int32),
        ]
        + [pltpu.VMEM((BB, HW), jnp.float32) for _ in range(DEPTH)]
        + [
            pltpu.VMEM((BB,), jnp.float32),
            pltpu.VMEM_SHARED((N_PAD, HW), jnp.float32),
            pltpu.VMEM_SHARED((N_PAD,), jnp.float32),
        ]
        + [pltpu.SemaphoreType.DMA for _ in range(2 * DEPTH + 1)]
    )

    def body(tables, srcs, dsts, *rest):
        if with_deg:
            acc_out, deg_out = rest[0], rest[1]
            rest = rest[2:]
        else:
            acc_out = rest[0]
            rest = rest[1:]
        src_v, dst_v = rest[0], rest[1]
        bufs = list(rest[2:2 + DEPTH])
        ones_v, acc_sh, deg_sh = rest[2 + DEPTH:5 + DEPTH]
        gsems = list(rest[5 + DEPTH:5 + 2 * DEPTH])
        ssems = list(rest[5 + 2 * DEPTH:5 + 3 * DEPTH])
        dsem = rest[5 + 3 * DEPTH]
        rows0 = bufs[0]
        cid = lax.axis_index("c")
        sid = lax.axis_index("s")
        base = sid * RPT
        on_core0 = cid == 0

        def zrow(r, carry):
            for c in range(HW // L):
                rows0[r, pl.ds(c * L, L)] = jnp.zeros((L,), jnp.float32)
            return carry

        lax.fori_loop(0, BB, zrow, 0)
        for k in range(RPT // BB):
            pltpu.sync_copy(rows0, acc_sh.at[pl.ds(base + k * BB, BB)])
        if with_deg:
            @pl.when(on_core0)
            def _():
                for k in range(RPT // HW):
                    pltpu.sync_copy(rows0.at[0],
                                    deg_sh.at[pl.ds(base + k * HW, HW)])
                for i in range(BB // L):
                    ones_v[pl.ds(i * L, L)] = jnp.ones((L,), jnp.float32)
        pltpu.sync_copy(srcs.at[sid], src_v)
        pltpu.sync_copy(dsts.at[sid], dst_v)
        plsc.subcore_barrier()

        for b in range(DEPTH):
            pltpu.async_copy(tables.at[cid].at[src_v.at[b]], bufs[b], gsems[b])

        def group(g, carry):
            for b in range(DEPTH):
                j = DEPTH * g + b
                pltpu.make_async_copy(tables.at[cid].at[src_v.at[j]],
                                      bufs[b], gsems[b]).wait()
                pltpu.async_copy(bufs[b], acc_sh.at[dst_v.at[j]], ssems[b],
                                 add=True)
                if with_deg:
                    @pl.when(on_core0)
                    def _():
                        pltpu.async_copy(ones_v, deg_sh.at[dst_v.at[j]], dsem,
                                         add=True)
            for b in range(DEPTH):
                jn = DEPTH * (g + 1) + b

                @pl.when(jn < NBC)
                def _():
                    pltpu.make_async_copy(bufs[b], acc_sh.at[dst_v.at[0]],
                                          ssems[b]).wait()
                    pltpu.async_copy(tables.at[cid].at[src_v.at[jn]],
                                     bufs[b], gsems[b])
            return carry

        lax.fori_loop(0, NBC // DEPTH, group, 0)
        for b in range(DEPTH):
            pltpu.make_async_copy(bufs[b], acc_sh.at[dst_v.at[0]],
                                  ssems[b]).wait()
        if with_deg:
            @pl.when(on_core0)
            def _():
                def drain(i, carry):
                    pltpu.make_async_copy(ones_v, deg_sh.at[dst_v.at[0]],
                                          dsem).wait()
                    return carry

                lax.fori_loop(0, NBC, drain, 0)
        plsc.subcore_barrier()

        pltpu.sync_copy(acc_sh.at[pl.ds(base, RPT)],
                        acc_out.at[pl.ds(base, RPT), cid])
        if with_deg:
            @pl.when(on_core0)
            def _():
                pltpu.sync_copy(deg_sh.at[pl.ds(base, RPT)],
                                deg_out.at[pl.ds(base, RPT)])

    return functools.partial(
        pl.kernel, mesh=mesh, out_type=out_type, scratch_types=scratch,
        compiler_params=pltpu.CompilerParams(use_tc_tiling_on_sc=False),
    )(body)


_seg_sum_deg = _make_seg_sum(True)
_seg_sum = _make_seg_sum(False)

BM = 2000


def _layer1_body(x_ref, a_ref, d_ref, ws_ref, wn_ref, b_ref, wn2_ref,
                 h_ref, g_ref):
    deg = jnp.maximum(d_ref[...], 1.0)
    hn = a_ref[...] / deg
    h = jnp.dot(x_ref[...], ws_ref[...], preferred_element_type=jnp.float32)
    h = h + jnp.dot(hn, wn_ref[...], preferred_element_type=jnp.float32)
    h = jnp.maximum(h + b_ref[...], 0.0)
    h_ref[...] = h
    g = jnp.dot(h, wn2_ref[...], preferred_element_type=jnp.float32)
    g_ref[0] = g[:, :HW]
    g_ref[1] = g[:, HW:]


def _layer2_body(h_ref, a_ref, d_ref, ws_ref, b_ref, out_ref):
    deg = jnp.maximum(d_ref[...], 1.0)
    hn = a_ref[...] / deg
    out = jnp.dot(h_ref[...], ws_ref[...], preferred_element_type=jnp.float32)
    out_ref[...] = out + hn + b_ref[...]


def _row_spec(w):
    return pl.BlockSpec((BM, w), lambda i: (i, 0))


def _full_spec(r, c):
    return pl.BlockSpec((r, c), lambda i: (0, 0))


def kernel(features, edge_index, W_self1, W_neigh1, b1, W_self2, W_neigh2, b2):
    src = edge_index[0].astype(jnp.int32)
    dst = edge_index[1].astype(jnp.int32)
    pad = E_PAD - E
    src_p = jnp.concatenate([src, jnp.zeros((pad,), jnp.int32)]).reshape(NS, NBC, BB)
    dst_p = jnp.concatenate([dst, jnp.full((pad,), N, jnp.int32)]).reshape(NS, NBC, BB)

    tab1 = jnp.stack([features[:, :HW], features[:, HW:]])
    acc1, degp = _seg_sum_deg(tab1, src_p, dst_p)
    a1 = acc1.reshape(N_PAD, D_IN)
    d = degp.reshape(N_PAD, 1)

    h, g = pl.pallas_call(
        _layer1_body,
        grid=(N // BM,),
        in_specs=[_row_spec(D_IN), _row_spec(D_IN), _row_spec(1),
                  _full_spec(D_IN, D_HID), _full_spec(D_IN, D_HID),
                  _full_spec(1, D_HID), _full_spec(D_HID, D_OUT)],
        out_specs=[_row_spec(D_HID),
                   pl.BlockSpec((NC, BM, HW), lambda i: (0, i, 0))],
        out_shape=[jax.ShapeDtypeStruct((N, D_HID), jnp.float32),
                   jax.ShapeDtypeStruct((NC, N, HW), jnp.float32)],
    )(features, a1, d, W_self1, W_neigh1, b1.reshape(1, -1), W_neigh2)

    (acc2,) = _seg_sum(g, src_p, dst_p)
    a2 = acc2.reshape(N_PAD, D_OUT)

    out = pl.pallas_call(
        _layer2_body,
        grid=(N // BM,),
        in_specs=[_row_spec(D_HID), _row_spec(D_OUT), _row_spec(1),
                  _full_spec(D_HID, D_OUT), _full_spec(1, D_OUT)],
        out_specs=_row_spec(D_OUT),
        out_shape=jax.ShapeDtypeStruct((N, D_OUT), jnp.float32),
    )(h, a2, d, W_self2, b2.reshape(1, -1))

    return out


# speedup vs baseline: 1.6224x; 1.5970x over previous
import functools

import jax
import jax.numpy as jnp
from jax import lax
from jax.experimental import pallas as pl
from jax.experimental.pallas import tpu as pltpu
from jax.experimental.pallas import tpu_sc as plsc

N = 10000
E = 320000
D_IN = 128
D_HID = 256
D_OUT = 128

NC = 2
NS = 16
L = 16
HW = D_IN // NC

DEPTH = 4
BB = 128
NBC = 160
E_PAD = NS * NBC * BB
N_PAD = 10240
RPT = N_PAD // NS


def _make_seg_sum(with_deg, dtype):
    mesh = plsc.VectorSubcoreMesh(core_axis_name="c", subcore_axis_name="s")
    vl = L * 4 // jnp.dtype(dtype).itemsize
    out_type = [jax.ShapeDtypeStruct((N_PAD, NC, HW), dtype)]
    if with_deg:
        out_type.append(jax.ShapeDtypeStruct((N_PAD,), jnp.float32))
    scratch = (
        [
            pltpu.VMEM((NBC, BB), jnp.int32),
            pltpu.VMEM((NBC, BB), jnp.int32),
        ]
        + [pltpu.VMEM((BB, HW), dtype) for _ in range(DEPTH)]
        + [
            pltpu.VMEM((BB,), jnp.float32),
            pltpu.VMEM((HW,), jnp.float32),
            pltpu.VMEM_SHARED((N_PAD, HW), dtype),
            pltpu.VMEM_SHARED((N_PAD,), jnp.float32),
        ]
        + [pltpu.SemaphoreType.DMA for _ in range(2 * DEPTH + 1)]
    )

    def body(tables, srcs, dsts, *rest):
        if with_deg:
            acc_out, deg_out = rest[0], rest[1]
            rest = rest[2:]
        else:
            acc_out = rest[0]
            rest = rest[1:]
        src_v, dst_v = rest[0], rest[1]
        bufs = list(rest[2:2 + DEPTH])
        ones_v, zvec, acc_sh, deg_sh = rest[2 + DEPTH:6 + DEPTH]
        gsems = list(rest[6 + DEPTH:6 + 2 * DEPTH])
        ssems = list(rest[6 + 2 * DEPTH:6 + 3 * DEPTH])
        dsem = rest[6 + 3 * DEPTH]
        rows0 = bufs[0]
        cid = lax.axis_index("c")
        sid = lax.axis_index("s")
        base = sid * RPT
        on_core0 = cid == 0

        def zrow(r, carry):
            for c in range(HW // vl):
                rows0[r, pl.ds(c * vl, vl)] = jnp.zeros((vl,), dtype)
            return carry

        lax.fori_loop(0, BB, zrow, 0)
        for k in range(RPT // BB):
            pltpu.sync_copy(rows0, acc_sh.at[pl.ds(base + k * BB, BB)])
        if with_deg:
            @pl.when(on_core0)
            def _():
                for c in range(HW // L):
                    zvec[pl.ds(c * L, L)] = jnp.zeros((L,), jnp.float32)
                for k in range(RPT // HW):
                    pltpu.sync_copy(zvec,
                                    deg_sh.at[pl.ds(base + k * HW, HW)])
                for i in range(BB // L):
                    ones_v[pl.ds(i * L, L)] = jnp.ones((L,), jnp.float32)
        pltpu.sync_copy(srcs.at[sid], src_v)
        pltpu.sync_copy(dsts.at[sid], dst_v)
        plsc.subcore_barrier()

        for b in range(DEPTH):
            pltpu.async_copy(tables.at[cid].at[src_v.at[b]], bufs[b], gsems[b])

        def group(g, carry):
            for b in range(DEPTH):
                j = DEPTH * g + b
                pltpu.make_async_copy(tables.at[cid].at[src_v.at[j]],
                                      bufs[b], gsems[b]).wait()
                pltpu.async_copy(bufs[b], acc_sh.at[dst_v.at[j]], ssems[b],
                                 add=True)
                if with_deg:
                    @pl.when(on_core0)
                    def _():
                        pltpu.async_copy(ones_v, deg_sh.at[dst_v.at[j]], dsem,
                                         add=True)
            for b in range(DEPTH):
                jn = DEPTH * (g + 1) + b

                @pl.when(jn < NBC)
                def _():
                    pltpu.make_async_copy(bufs[b], acc_sh.at[dst_v.at[0]],
                                          ssems[b]).wait()
                    pltpu.async_copy(tables.at[cid].at[src_v.at[jn]],
                                     bufs[b], gsems[b])
            return carry

        lax.fori_loop(0, NBC // DEPTH, group, 0)
        for b in range(DEPTH):
            pltpu.make_async_copy(bufs[b], acc_sh.at[dst_v.at[0]],
                                  ssems[b]).wait()
        if with_deg:
            @pl.when(on_core0)
            def _():
                def drain(i, carry):
                    pltpu.make_async_copy(ones_v, deg_sh.at[dst_v.at[0]],
                                          dsem).wait()
                    return carry

                lax.fori_loop(0, NBC, drain, 0)
        plsc.subcore_barrier()

        pltpu.sync_copy(acc_sh.at[pl.ds(base, RPT)],
                        acc_out.at[pl.ds(base, RPT), cid])
        if with_deg:
            @pl.when(on_core0)
            def _():
                pltpu.sync_copy(deg_sh.at[pl.ds(base, RPT)],
                                deg_out.at[pl.ds(base, RPT)])

    return functools.partial(
        pl.kernel, mesh=mesh, out_type=out_type, scratch_types=scratch,
        compiler_params=pltpu.CompilerParams(use_tc_tiling_on_sc=False),
    )(body)


_AGG_DT = jnp.bfloat16
_seg_sum_deg = _make_seg_sum(True, _AGG_DT)
_seg_sum = _make_seg_sum(False, _AGG_DT)

BM = 2000


def _layer1_body(x_ref, a_ref, d_ref, ws_ref, wn_ref, b_ref, wn2_ref,
                 h_ref, g_ref):
    deg = jnp.maximum(d_ref[...], 1.0)
    hn = a_ref[...].astype(jnp.float32) / deg
    h = jnp.dot(x_ref[...], ws_ref[...], preferred_element_type=jnp.float32)
    h = h + jnp.dot(hn, wn_ref[...], preferred_element_type=jnp.float32)
    h = jnp.maximum(h + b_ref[...], 0.0)
    h_ref[...] = h
    g = jnp.dot(h, wn2_ref[...], preferred_element_type=jnp.float32)
    g_ref[0] = g[:, :HW].astype(_AGG_DT)
    g_ref[1] = g[:, HW:].astype(_AGG_DT)


def _layer2_body(h_ref, a_ref, d_ref, ws_ref, b_ref, out_ref):
    deg = jnp.maximum(d_ref[...], 1.0)
    hn = a_ref[...].astype(jnp.float32) / deg
    out = jnp.dot(h_ref[...], ws_ref[...], preferred_element_type=jnp.float32)
    out_ref[...] = out + hn + b_ref[...]


def _row_spec(w):
    return pl.BlockSpec((BM, w), lambda i: (i, 0))


def _full_spec(r, c):
    return pl.BlockSpec((r, c), lambda i: (0, 0))


def kernel(features, edge_index, W_self1, W_neigh1, b1, W_self2, W_neigh2, b2):
    src = edge_index[0].astype(jnp.int32)
    dst = edge_index[1].astype(jnp.int32)
    pad = E_PAD - E
    src_p = jnp.concatenate([src, jnp.zeros((pad,), jnp.int32)]).reshape(NS, NBC, BB)
    dst_p = jnp.concatenate([dst, jnp.full((pad,), N, jnp.int32)]).reshape(NS, NBC, BB)

    tab1 = jnp.stack([features[:, :HW], features[:, HW:]]).astype(_AGG_DT)
    acc1, degp = _seg_sum_deg(tab1, src_p, dst_p)
    a1 = acc1.reshape(N_PAD, D_IN)
    d = degp.reshape(N_PAD, 1)

    h, g = pl.pallas_call(
        _layer1_body,
        grid=(N // BM,),
        in_specs=[_row_spec(D_IN), _row_spec(D_IN), _row_spec(1),
                  _full_spec(D_IN, D_HID), _full_spec(D_IN, D_HID),
                  _full_spec(1, D_HID), _full_spec(D_HID, D_OUT)],
        out_specs=[_row_spec(D_HID),
                   pl.BlockSpec((NC, BM, HW), lambda i: (0, i, 0))],
        out_shape=[jax.ShapeDtypeStruct((N, D_HID), jnp.float32),
                   jax.ShapeDtypeStruct((NC, N, HW), _AGG_DT)],
    )(features, a1, d, W_self1, W_neigh1, b1.reshape(1, -1), W_neigh2)

    (acc2,) = _seg_sum(g, src_p, dst_p)
    a2 = acc2.reshape(N_PAD, D_OUT)

    out = pl.pallas_call(
        _layer2_body,
        grid=(N // BM,),
        in_specs=[_row_spec(D_HID), _row_spec(D_OUT), _row_spec(1),
                  _full_spec(D_HID, D_OUT), _full_spec(1, D_OUT)],
        out_specs=_row_spec(D_OUT),
        out_shape=jax.ShapeDtypeStruct((N, D_OUT), jnp.float32),
    )(h, a2, d, W_self2, b2.reshape(1, -1))

    return out
